# baseline (speedup 1.0000x reference)
import jax
import jax.numpy as jnp
from jax import lax
from jax.experimental import pallas as pl
from jax.experimental.pallas import tpu as pltpu
from jax.experimental.pallas import tpu_sc as plsc

VOCAB = 1000000
EMB_DIM = 64
BATCH = 1024
SEQ = 200

NW = 32
ROWS_PER_W = BATCH // NW
W0 = 128
W1 = SEQ - W0
NBUF = 4


def _gather_fn():
    mesh = plsc.VectorSubcoreMesh(
        core_axis_name="core", subcore_axis_name="subcore"
    )

    @jax.jit
    def gather(table, idx):
        @pl.kernel(
            out_type=jax.ShapeDtypeStruct((BATCH, SEQ, EMB_DIM), table.dtype),
            mesh=mesh,
            scratch_types=[
                pltpu.VMEM((ROWS_PER_W, SEQ), jnp.int32),
                pltpu.VMEM((NBUF, SEQ, 2 * EMB_DIM), jnp.float32),
                pltpu.SemaphoreType.DMA,
                pltpu.SemaphoreType.DMA((NBUF,)),
                pltpu.SemaphoreType.DMA((NBUF,)),
            ],
            compiler_params=pltpu.CompilerParams(use_tc_tiling_on_sc=False),
        )
        def kernel(x_hbm, i_hbm, o_hbm, idx_v, rows_v, isem, gsem, ssem):
            cid = lax.axis_index("core")
            sid = lax.axis_index("subcore")
            wid = sid * 2 + cid
            row0 = wid * ROWS_PER_W

            pltpu.async_copy(
                i_hbm.at[pl.ds(row0, ROWS_PER_W)], idx_v, isem
            ).wait()

            def start_row(r, b):
                pltpu.async_copy(
                    x_hbm.at[idx_v.at[r, pl.ds(0, W0)]],
                    rows_v.at[b, pl.ds(0, W0)],
                    gsem.at[b],
                )
                pltpu.async_copy(
                    x_hbm.at[idx_v.at[r, pl.ds(W0, W1)]],
                    rows_v.at[b, pl.ds(W0, W1)],
                    gsem.at[b],
                )

            def wait_row(r, b):
                pltpu.make_async_copy(
                    x_hbm.at[idx_v.at[r, pl.ds(0, W0)]],
                    rows_v.at[b, pl.ds(0, W0)],
                    gsem.at[b],
                ).wait()
                pltpu.make_async_copy(
                    x_hbm.at[idx_v.at[r, pl.ds(W0, W1)]],
                    rows_v.at[b, pl.ds(W0, W1)],
                    gsem.at[b],
                ).wait()

            for b in range(NBUF):
                start_row(b, b)

            @pl.loop(0, ROWS_PER_W, step=NBUF)
            def _(r0):
                for b in range(NBUF):
                    r = r0 + b
                    wait_row(r, b)
                    out_slice = o_hbm.at[row0 + r]
                    src = rows_v.at[b, :, pl.ds(0, EMB_DIM)]
                    pltpu.async_copy(src, out_slice, ssem.at[b])
                    pltpu.make_async_copy(src, out_slice, ssem.at[b]).wait()

                    @pl.when(r + NBUF < ROWS_PER_W)
                    def _():
                        start_row(r + NBUF, b)

        return kernel(table, idx)

    return gather


_gather = _gather_fn()


def kernel(inp, emb_weight):
    idx = jnp.clip(inp, 0, VOCAB - 1)
    tpad = jnp.pad(emb_weight, ((0, 0), (0, EMB_DIM)))
    return _gather(tpad, idx)
